# Initial kernel scaffold; baseline (speedup 1.0000x reference)
import jax
import jax.numpy as jnp
from jax import lax
from jax.experimental import pallas as pl
from jax.experimental.pallas import tpu as pltpu

_E = 16
_TOPK = 2
_HID = 1024
_FF = 2048
_T = 128
_NP = _T * _TOPK
_R = 32
_PADR = _NP + _R
_FBLK = 1024
_NF = _FF // _FBLK


def _moe_body(se_ref, w_ref, x_ref, gate_ref, up_ref, down_ref, out_ref,
              xs_ref, o_ref, p_ref, es_ref, ws_ref):
    e = pl.program_id(0)
    f = pl.program_id(1)

    @pl.when(jnp.logical_and(e == 0, f == 0))
    def _prologue():
        se_row = se_ref[...]
        idx_row = lax.broadcasted_iota(jnp.int32, (1, _NP), 1)
        key_row = (se_row * _NP + idx_row).astype(jnp.float32)
        ii = lax.broadcasted_iota(jnp.float32, (_NP, _NP), 0)
        jj = lax.broadcasted_iota(jnp.float32, (_NP, _NP), 1)
        eye = (ii == jj).astype(jnp.float32)
        key_col = lax.dot_general(eye, key_row, (((1,), (1,)), ((), ())),
                                  preferred_element_type=jnp.float32)
        ltm = (jnp.broadcast_to(key_row, (_NP, _NP)) < key_col).astype(jnp.float32)
        rank_col = jnp.sum(ltm, axis=1, keepdims=True)
        rank_row = lax.dot_general(rank_col, eye, (((0,), (0,)), ((), ())),
                                   preferred_element_type=jnp.float32)
        bmat = (jnp.broadcast_to(rank_row, (_NP, _NP)) == ii).astype(jnp.float32)
        tok_row = (idx_row // _TOPK).astype(jnp.float32)
        tok_sorted = lax.dot_general(bmat, tok_row, (((1,), (1,)), ((), ())),
                                     preferred_element_type=jnp.float32)
        w_sorted = lax.dot_general(bmat, w_ref[...], (((1,), (1,)), ((), ())),
                                   preferred_element_type=jnp.float32)
        e_sorted = lax.dot_general(bmat, se_row.astype(jnp.float32),
                                   (((1,), (1,)), ((), ())),
                                   preferred_element_type=jnp.float32)
        tcols = lax.broadcasted_iota(jnp.float32, (_NP, _T), 1)
        p_ref[...] = (jnp.broadcast_to(tok_sorted, (_NP, _T)) == tcols).astype(
            jnp.float32)
        xs_ref[0:_NP, :] = jnp.dot(p_ref[...], x_ref[...],
                                   preferred_element_type=jnp.float32)
        xs_ref[_NP:_PADR, :] = jnp.zeros((_R, _HID), jnp.float32)
        o_ref[...] = jnp.zeros((_PADR, _HID), jnp.float32)
        es_ref[0:_NP, :] = jnp.broadcast_to(e_sorted, (_NP, 128))
        es_ref[_NP:_PADR, :] = -jnp.ones((_R, 128), jnp.float32)
        ws_ref[...] = jnp.broadcast_to(w_sorted, (_NP, 128))

    se_all = se_ref[...]
    cnt = jnp.sum((se_all == e).astype(jnp.int32))
    start = jnp.sum((se_all < e).astype(jnp.int32))
    ntiles = (cnt + _R - 1) // _R
    gate = gate_ref[...]
    up = up_ref[...]
    down = down_ref[...]
    e_f32 = e.astype(jnp.float32)

    def _tile(j, carry):
        base = start + j * _R
        tile = xs_ref[pl.ds(base, _R), :]
        g = jnp.dot(tile, gate, preferred_element_type=jnp.float32)
        u = jnp.dot(tile, up, preferred_element_type=jnp.float32)
        h = jax.nn.gelu(g, approximate=True) * u
        o = jnp.dot(h, down, preferred_element_type=jnp.float32)
        m = (es_ref[pl.ds(base, _R), 0:1] == e_f32).astype(jnp.float32)
        o_ref[pl.ds(base, _R), :] += o * m
        return carry

    lax.fori_loop(0, ntiles, _tile, 0)

    @pl.when(jnp.logical_and(e == _E - 1, f == _NF - 1))
    def _epilogue():
        o_scaled = o_ref[0:_NP, :] * ws_ref[:, 0:1]
        out_ref[...] = lax.dot_general(p_ref[...], o_scaled,
                                       (((0,), (0,)), ((), ())),
                                       preferred_element_type=jnp.float32)


def _moe_call(se, rw, x, gate_w, up_w, down_w, interpret=False):
    return pl.pallas_call(
        _moe_body,
        grid=(_E, _NF),
        in_specs=[
            pl.BlockSpec((1, _NP), lambda e, f: (0, 0)),
            pl.BlockSpec((1, _NP), lambda e, f: (0, 0)),
            pl.BlockSpec((_T, _HID), lambda e, f: (0, 0)),
            pl.BlockSpec((None, _HID, _FBLK), lambda e, f: (e, 0, f)),
            pl.BlockSpec((None, _HID, _FBLK), lambda e, f: (e, 0, f)),
            pl.BlockSpec((None, _FBLK, _HID), lambda e, f: (e, f, 0)),
        ],
        out_specs=pl.BlockSpec((_T, _HID), lambda e, f: (0, 0)),
        out_shape=jax.ShapeDtypeStruct((_T, _HID), jnp.float32),
        scratch_shapes=[
            pltpu.VMEM((_PADR, _HID), jnp.float32),
            pltpu.VMEM((_PADR, _HID), jnp.float32),
            pltpu.VMEM((_NP, _T), jnp.float32),
            pltpu.VMEM((_PADR, 128), jnp.float32),
            pltpu.VMEM((_NP, 128), jnp.float32),
        ],
        compiler_params=pltpu.CompilerParams(
            dimension_semantics=("arbitrary", "arbitrary"),
        ),
        interpret=interpret,
    )(se, rw, x, gate_w, up_w, down_w)


def kernel(x, selected_experts, routing_weights, gate_w, up_w, down_w):
    se = selected_experts.reshape(1, _NP).astype(jnp.int32)
    rw = routing_weights.reshape(1, _NP).astype(jnp.float32)
    return _moe_call(se, rw, x, gate_w, up_w, down_w)


# speedup vs baseline: 2.2677x; 2.2677x over previous
import jax
import jax.numpy as jnp
from jax import lax
from jax.experimental import pallas as pl
from jax.experimental.pallas import tpu as pltpu

_E = 16
_TOPK = 2
_HID = 1024
_FF = 2048
_T = 128
_NP = _T * _TOPK
_R = 32
_PADR = _NP + _R
_FBLK = 1024
_NF = _FF // _FBLK


def _moe_body(se_ref, w_ref, x_ref, gate_ref, up_ref, down_ref, out_ref,
              xs_ref, o_ref, p_ref, es_ref, ws_ref):
    e = pl.program_id(0)
    f = pl.program_id(1)

    @pl.when(jnp.logical_and(e == 0, f == 0))
    def _prologue():
        se_row = se_ref[...]
        idx_row = lax.broadcasted_iota(jnp.int32, (1, _NP), 1)
        key_row = (se_row * _NP + idx_row).astype(jnp.float32)
        ii = lax.broadcasted_iota(jnp.int32, (_NP, _NP), 0).astype(jnp.float32)
        jj = lax.broadcasted_iota(jnp.int32, (_NP, _NP), 1).astype(jnp.float32)
        eye = (ii == jj).astype(jnp.float32)
        key_col = lax.dot_general(eye, key_row, (((1,), (1,)), ((), ())),
                                  preferred_element_type=jnp.float32)
        ltm = (jnp.broadcast_to(key_row, (_NP, _NP)) < key_col).astype(jnp.float32)
        rank_col = jnp.sum(ltm, axis=1, keepdims=True)
        rank_row = lax.dot_general(rank_col, eye, (((0,), (0,)), ((), ())),
                                   preferred_element_type=jnp.float32)
        bmat = (jnp.broadcast_to(rank_row, (_NP, _NP)) == ii).astype(jnp.float32)
        tok_row = (idx_row // _TOPK).astype(jnp.float32)
        tok_sorted = lax.dot_general(bmat, tok_row, (((1,), (1,)), ((), ())),
                                     preferred_element_type=jnp.float32)
        w_sorted = lax.dot_general(bmat, w_ref[...], (((1,), (1,)), ((), ())),
                                   preferred_element_type=jnp.float32)
        e_sorted = lax.dot_general(bmat, se_row.astype(jnp.float32),
                                   (((1,), (1,)), ((), ())),
                                   preferred_element_type=jnp.float32)
        tcols = lax.broadcasted_iota(jnp.int32, (_NP, _T), 1).astype(jnp.float32)
        p_ref[...] = (jnp.broadcast_to(tok_sorted, (_NP, _T)) == tcols).astype(
            jnp.float32)
        xs_ref[0:_NP, :] = jnp.dot(p_ref[...], x_ref[...],
                                   preferred_element_type=jnp.float32)
        xs_ref[_NP:_PADR, :] = jnp.zeros((_R, _HID), jnp.float32)
        o_ref[...] = jnp.zeros((_PADR, _HID), jnp.float32)
        es_ref[0:_NP, :] = jnp.broadcast_to(e_sorted, (_NP, 128))
        es_ref[_NP:_PADR, :] = -jnp.ones((_R, 128), jnp.float32)
        ws_ref[...] = jnp.broadcast_to(w_sorted, (_NP, 128))

    se_all = se_ref[...]
    cnt = jnp.sum((se_all == e).astype(jnp.int32))
    start = jnp.sum((se_all < e).astype(jnp.int32))
    astart = (start // 8) * 8
    ntiles = ((start - astart) + cnt + _R - 1) // _R
    gate = gate_ref[...]
    up = up_ref[...]
    down = down_ref[...]
    e_f32 = e.astype(jnp.float32)

    def _tile(j, carry):
        base = pl.multiple_of(astart + j * _R, 8)
        tile = xs_ref[pl.ds(base, _R), :]
        g = jnp.dot(tile, gate, preferred_element_type=jnp.float32)
        u = jnp.dot(tile, up, preferred_element_type=jnp.float32)
        h = jax.nn.gelu(g, approximate=True) * u
        o = jnp.dot(h, down, preferred_element_type=jnp.float32)
        m = (es_ref[pl.ds(base, _R), 0:1] == e_f32).astype(jnp.float32)
        o_ref[pl.ds(base, _R), :] += o * m
        return carry

    lax.fori_loop(0, ntiles, _tile, 0)

    @pl.when(jnp.logical_and(e == _E - 1, f == _NF - 1))
    def _epilogue():
        o_scaled = o_ref[0:_NP, :] * ws_ref[:, 0:1]
        out_ref[...] = lax.dot_general(p_ref[...], o_scaled,
                                       (((0,), (0,)), ((), ())),
                                       preferred_element_type=jnp.float32)


def _moe_call(se, rw, x, gate_w, up_w, down_w, interpret=False):
    return pl.pallas_call(
        _moe_body,
        grid=(_E, _NF),
        in_specs=[
            pl.BlockSpec((1, _NP), lambda e, f: (0, 0)),
            pl.BlockSpec((1, _NP), lambda e, f: (0, 0)),
            pl.BlockSpec((_T, _HID), lambda e, f: (0, 0)),
            pl.BlockSpec((None, _HID, _FBLK), lambda e, f: (e, 0, f)),
            pl.BlockSpec((None, _HID, _FBLK), lambda e, f: (e, 0, f)),
            pl.BlockSpec((None, _FBLK, _HID), lambda e, f: (e, f, 0)),
        ],
        out_specs=pl.BlockSpec((_T, _HID), lambda e, f: (0, 0)),
        out_shape=jax.ShapeDtypeStruct((_T, _HID), jnp.float32),
        scratch_shapes=[
            pltpu.VMEM((_PADR, _HID), jnp.float32),
            pltpu.VMEM((_PADR, _HID), jnp.float32),
            pltpu.VMEM((_NP, _T), jnp.float32),
            pltpu.VMEM((_PADR, 128), jnp.float32),
            pltpu.VMEM((_NP, 128), jnp.float32),
        ],
        compiler_params=pltpu.CompilerParams(
            dimension_semantics=("arbitrary", "arbitrary"),
        ),
        interpret=interpret,
    )(se, rw, x, gate_w, up_w, down_w)


def kernel(x, selected_experts, routing_weights, gate_w, up_w, down_w):
    se = selected_experts.reshape(1, _NP).astype(jnp.int32)
    rw = routing_weights.reshape(1, _NP).astype(jnp.float32)
    return _moe_call(se, rw, x, gate_w, up_w, down_w)
